# Initial kernel scaffold; baseline (speedup 1.0000x reference)
import jax
import jax.numpy as jnp
from jax.experimental import pallas as pl


def kernel(x, edge_index, W1, b1, W2, b2):
    raise NotImplementedError("write your pallas kernel here")


# speedup vs baseline: 4.4429x; 4.4429x over previous
import functools

import jax
import jax.numpy as jnp
from jax import lax
from jax.experimental import pallas as pl
from jax.experimental.pallas import tpu as pltpu
from jax.experimental.pallas import tpu_sc as plsc

N = 10000
NPAD = 10240
RPT = 640
RC = 128
NRC = RPT // RC
D = 256
DQ = 64
E = 160000
EC = 128
NCHUNK = 80
EPT = NCHUNK * EC
R_HOPS = 3
NSUB = 16
LANES = 16
ZR = 32


def _sc_body(x_qsplit, srcp, dstp, hops, gflat, degpart,
             src_v, dst_v, gbufv, rowbuf, gout, zbuf, deglocal,
             degsum, disv, acc, sem, sem2):
    s = lax.axis_index("s")
    c = lax.axis_index("c")
    nbase = s * RPT

    zeros16 = jnp.zeros((LANES,), jnp.float32)
    ones16 = jnp.ones((LANES,), jnp.float32)

    def _zrow(i, _):
        for g in range(DQ // LANES):
            zbuf[i, pl.ds(g * LANES, LANES)] = zeros16
        return 0
    lax.fori_loop(0, ZR, _zrow, 0)

    def _zero_acc(rbase):
        for z in range(RC // ZR):
            pltpu.sync_copy(zbuf, acc.at[pl.ds(rbase + z * ZR, ZR)])

    def _dz(i, _):
        deglocal[pl.ds(i * LANES, LANES)] = zeros16
        return 0
    lax.fori_loop(0, NPAD // LANES, _dz, 0)

    for cc in range(NRC):
        _zero_acc(nbase + cc * RC)

    pltpu.sync_copy(srcp.at[pl.ds(s * NCHUNK, NCHUNK)], src_v.at[pl.ds(0, NCHUNK)])
    pltpu.sync_copy(srcp.at[pl.ds(s * NCHUNK, NCHUNK)], src_v.at[pl.ds(NCHUNK, NCHUNK)])
    pltpu.sync_copy(dstp.at[pl.ds(s * NCHUNK, NCHUNK)], dst_v)

    def _bias(j, _):
        for g in range(EC // LANES):
            sl = pl.ds(g * LANES, LANES)
            src_v[j, sl] = src_v[j, sl] + (2 * c) * NPAD
            src_v[NCHUNK + j, sl] = src_v[NCHUNK + j, sl] + (2 * c + 1) * NPAD
        return 0
    lax.fori_loop(0, NCHUNK, _bias, 0)

    def _deg(j, _):
        for g in range(EC // LANES):
            dvals = dst_v[j, pl.ds(g * LANES, LANES)]
            plsc.addupdate_scatter(deglocal, [dvals], ones16)
        return 0
    lax.fori_loop(0, NCHUNK, _deg, 0)

    pltpu.sync_copy(deglocal, degpart.at[c, s])
    plsc.subcore_barrier()
    pltpu.sync_copy(degpart.at[c, :, pl.ds(nbase, RPT)], degsum)

    def _dis(gg, _):
        sl = pl.ds(gg * LANES, LANES)
        dvec = degsum[0, sl]
        for t in range(1, NSUB):
            dvec = dvec + degsum[t, sl]
        d1 = jnp.maximum(dvec, 1.0)
        bits = lax.bitcast_convert_type(d1, jnp.int32)
        bits = jnp.int32(0x5F3759DF) - lax.shift_right_arithmetic(bits, 1)
        y = lax.bitcast_convert_type(bits, jnp.float32)
        for _ in range(4):
            y = y * (1.5 - 0.5 * d1 * y * y)
        disv[sl] = jnp.where(dvec > 0.5, y, 0.0)
        return 0
    lax.fori_loop(0, RPT // LANES, _dis, 0)

    def _scale_rows(coff, also_g):
        def _grp(gg, _):
            dvec = disv[pl.ds(coff + gg * LANES, LANES)]
            for i in range(LANES):
                dv = jnp.broadcast_to(dvec[i], (LANES,))
                row = gg * LANES + i
                for g in range(DQ // LANES):
                    sl = pl.ds(g * LANES, LANES)
                    v = rowbuf[row, sl] * dv
                    rowbuf[row, sl] = v
                    if also_g:
                        gout[row, sl] = v * dv
            return 0
        lax.fori_loop(0, RC // LANES, _grp, 0)

    for q in range(2):
        qid = 2 * c + q

        def _g0(cc, _):
            rbase = nbase + cc * RC
            pltpu.sync_copy(x_qsplit.at[qid, pl.ds(rbase, RC)], rowbuf)
            _scale_rows(cc * RC, False)
            pltpu.sync_copy(rowbuf, gflat.at[pl.ds(qid * NPAD + rbase, RC)])
            return 0
        lax.fori_loop(0, NRC, _g0, 0)

    plsc.subcore_barrier()

    def _pass(p, _):
        k = p // 2
        q = p % 2
        qid = 2 * c + q

        pltpu.async_copy(gflat.at[src_v.at[q * NCHUNK]], gbufv.at[0],
                         sem.at[0])

        def _edge(j, _):
            b = lax.rem(j, 2)
            pltpu.make_async_copy(
                gflat.at[src_v.at[q * NCHUNK + j]], gbufv.at[b],
                sem.at[b]).wait()

            @pl.when(j >= 1)
            def _():
                pltpu.make_async_copy(
                    gbufv.at[1 - b], acc.at[dst_v.at[j - 1]],
                    sem2.at[1 - b]).wait()

            @pl.when(j < NCHUNK - 1)
            def _():
                pltpu.async_copy(
                    gflat.at[src_v.at[q * NCHUNK + j + 1]], gbufv.at[1 - b],
                    sem.at[1 - b])
            pltpu.async_copy(gbufv.at[b], acc.at[dst_v.at[j]], sem2.at[b],
                             add=True)
            return 0
        lax.fori_loop(0, NCHUNK, _edge, 0)
        pltpu.make_async_copy(
            gbufv.at[(NCHUNK - 1) % 2], acc.at[dst_v.at[NCHUNK - 1]],
            sem2.at[(NCHUNK - 1) % 2]).wait()

        plsc.subcore_barrier()

        def _scale(cc, _):
            rbase = nbase + cc * RC
            pltpu.sync_copy(acc.at[pl.ds(rbase, RC)], rowbuf)
            _scale_rows(cc * RC, True)
            pltpu.sync_copy(rowbuf, hops.at[k, qid, pl.ds(rbase, RC)])

            @pl.when(k < R_HOPS - 1)
            def _():
                pltpu.sync_copy(gout, gflat.at[pl.ds(qid * NPAD + rbase, RC)])
            _zero_acc(rbase)
            return 0
        lax.fori_loop(0, NRC, _scale, 0)

        plsc.subcore_barrier()
        return 0
    lax.fori_loop(0, 2 * R_HOPS, _pass, 0)


def _sc_propagate(x_qsplit, srcp, dstp):
    mesh = plsc.VectorSubcoreMesh(
        core_axis_name="c", subcore_axis_name="s",
        num_cores=2, num_subcores=NSUB)
    return pl.kernel(
        _sc_body,
        out_type=[
            jax.ShapeDtypeStruct((R_HOPS, 4, NPAD, DQ), jnp.float32),
            jax.ShapeDtypeStruct((4 * NPAD, DQ), jnp.float32),
            jax.ShapeDtypeStruct((2, NSUB, NPAD), jnp.float32),
        ],
        mesh=mesh,
        scratch_types=[
            pltpu.VMEM((2 * NCHUNK, EC), jnp.int32),
            pltpu.VMEM((NCHUNK, EC), jnp.int32),
            pltpu.VMEM((2, EC, DQ), jnp.float32),
            pltpu.VMEM((RC, DQ), jnp.float32),
            pltpu.VMEM((RC, DQ), jnp.float32),
            pltpu.VMEM((ZR, DQ), jnp.float32),
            pltpu.VMEM((NPAD,), jnp.float32),
            pltpu.VMEM((NSUB, RPT), jnp.float32),
            pltpu.VMEM((RPT,), jnp.float32),
            pltpu.VMEM_SHARED((NPAD, DQ), jnp.float32),
            pltpu.SemaphoreType.DMA((2,)),
            pltpu.SemaphoreType.DMA((2,)),
        ],
        compiler_params=pltpu.CompilerParams(
            needs_layout_passes=False, use_tc_tiling_on_sc=False),
    )(x_qsplit, srcp, dstp)


def _mlp_body(x_ref, hops_ref, w1_ref, b1_ref, w2_ref, b2_ref, o_ref):
    acc = jnp.dot(x_ref[...], w1_ref[0:D, :],
                  preferred_element_type=jnp.float32)
    for r in range(R_HOPS):
        for qd in range(4):
            base = D + r * D + qd * DQ
            acc += jnp.dot(hops_ref[r, qd], w1_ref[base:base + DQ, :],
                           preferred_element_type=jnp.float32)
    hid = jnp.maximum(acc + b1_ref[...], 0.0)
    o_ref[...] = jnp.dot(hid, w2_ref[...],
                         preferred_element_type=jnp.float32) + b2_ref[...]


def _mlp(x_pad, hops, W1, b1, W2, b2):
    BR = 1024
    grid = (NPAD // BR,)
    return pl.pallas_call(
        _mlp_body,
        grid=grid,
        in_specs=[
            pl.BlockSpec((BR, D), lambda i: (i, 0)),
            pl.BlockSpec((R_HOPS, 4, BR, DQ), lambda i: (0, 0, i, 0)),
            pl.BlockSpec((D * (R_HOPS + 1), D), lambda i: (0, 0)),
            pl.BlockSpec((1, D), lambda i: (0, 0)),
            pl.BlockSpec((D, D), lambda i: (0, 0)),
            pl.BlockSpec((1, D), lambda i: (0, 0)),
        ],
        out_specs=pl.BlockSpec((BR, D), lambda i: (i, 0)),
        out_shape=jax.ShapeDtypeStruct((NPAD, D), jnp.float32),
        compiler_params=pltpu.CompilerParams(
            dimension_semantics=("arbitrary",),
        ),
    )(x_pad, hops, W1, b1, W2, b2)


@jax.jit
def kernel(x, edge_index, W1, b1, W2, b2):
    x_pad = jnp.pad(x, ((0, NPAD - N), (0, 0)))
    x_qsplit = x_pad.reshape(NPAD, 4, DQ).transpose(1, 0, 2)

    ei = edge_index.astype(jnp.int32)
    pad_e = NSUB * EPT - E
    src = jnp.pad(ei[0], (0, pad_e), constant_values=N)
    dst = jnp.pad(ei[1], (0, pad_e), constant_values=N)
    srcp = src.reshape(NSUB * NCHUNK, EC)
    dstp = dst.reshape(NSUB * NCHUNK, EC)

    hops, _g, _dp = _sc_propagate(x_qsplit, srcp, dstp)

    out = _mlp(x_pad, hops, W1, b1.reshape(1, D), W2, b2.reshape(1, D))
    return out[:N]


# speedup vs baseline: 5.2126x; 1.1732x over previous
import functools

import jax
import jax.numpy as jnp
from jax import lax
from jax.experimental import pallas as pl
from jax.experimental.pallas import tpu as pltpu
from jax.experimental.pallas import tpu_sc as plsc

N = 10000
NPAD = 10240
RPT = 640
RC = 128
NRC = RPT // RC
D = 256
DQ = 64
E = 160000
EC = 128
NCHUNK = 80
EPT = NCHUNK * EC
R_HOPS = 3
NSUB = 16
LANES = 16
ZR = 32
RD = 4


def _sc_body(x_qsplit, srcp, dstp, hops, gflat, degpart,
             src_v, dst_v, gbufv, rowbuf, gout, zbuf, deglocal,
             degsum, disv, acc, sem, sem2):
    s = lax.axis_index("s")
    c = lax.axis_index("c")
    nbase = s * RPT

    zeros16 = jnp.zeros((LANES,), jnp.float32)
    ones16 = jnp.ones((LANES,), jnp.float32)

    def _zrow(i, _):
        for g in range(DQ // LANES):
            zbuf[i, pl.ds(g * LANES, LANES)] = zeros16
        return 0
    lax.fori_loop(0, ZR, _zrow, 0)

    def _zero_acc(rbase):
        for z in range(RC // ZR):
            pltpu.sync_copy(zbuf, acc.at[pl.ds(rbase + z * ZR, ZR)])

    def _dz(i, _):
        deglocal[pl.ds(i * LANES, LANES)] = zeros16
        return 0
    lax.fori_loop(0, NPAD // LANES, _dz, 0)

    for cc in range(NRC):
        _zero_acc(nbase + cc * RC)

    pltpu.sync_copy(srcp.at[pl.ds(s * NCHUNK, NCHUNK)], src_v)
    pltpu.sync_copy(dstp.at[pl.ds(s * NCHUNK, NCHUNK)], dst_v)

    def _bias(j, _):
        for g in range(EC // LANES):
            sl = pl.ds(g * LANES, LANES)
            src_v[j, sl] = src_v[j, sl] + (2 * c) * NPAD
        return 0
    lax.fori_loop(0, NCHUNK, _bias, 0)

    def _deg(j, _):
        for g in range(EC // LANES):
            dvals = dst_v[j, pl.ds(g * LANES, LANES)]
            plsc.addupdate_scatter(deglocal, [dvals], ones16)
        return 0
    lax.fori_loop(0, NCHUNK, _deg, 0)

    pltpu.sync_copy(deglocal, degpart.at[c, s])
    plsc.subcore_barrier()

    def _dis_chunk(cc, _):
        pltpu.sync_copy(degpart.at[c, :, pl.ds(nbase + cc * RC, RC)], degsum)

        def _dis(gg, _):
            sl = pl.ds(gg * LANES, LANES)
            dvec = degsum[0, sl]
            for t in range(1, NSUB):
                dvec = dvec + degsum[t, sl]
            d1 = jnp.maximum(dvec, 1.0)
            bits = lax.bitcast_convert_type(d1, jnp.int32)
            bits = jnp.int32(0x5F3759DF) - lax.shift_right_arithmetic(bits, 1)
            y = lax.bitcast_convert_type(bits, jnp.float32)
            for _ in range(4):
                y = y * (1.5 - 0.5 * d1 * y * y)
            disv[pl.ds(cc * RC + gg * LANES, LANES)] = jnp.where(
                dvec > 0.5, y, 0.0)
            return 0
        lax.fori_loop(0, RC // LANES, _dis, 0)
        return 0
    lax.fori_loop(0, NRC, _dis_chunk, 0)

    def _scale_rows(coff, also_g):
        def _grp(gg, _):
            dvec = disv[pl.ds(coff + gg * LANES, LANES)]
            for i in range(LANES):
                dv = jnp.broadcast_to(dvec[i], (LANES,))
                row = gg * LANES + i
                for g in range(DQ // LANES):
                    sl = pl.ds(g * LANES, LANES)
                    v = rowbuf[row, sl] * dv
                    rowbuf[row, sl] = v
                    if also_g:
                        gout[row, sl] = v * dv
            return 0
        lax.fori_loop(0, RC // LANES, _grp, 0)

    for q in range(2):
        qid = 2 * c + q

        def _g0(cc, _):
            rbase = nbase + cc * RC
            pltpu.sync_copy(x_qsplit.at[qid, pl.ds(rbase, RC)], rowbuf)
            _scale_rows(cc * RC, False)
            pltpu.sync_copy(rowbuf, gflat.at[pl.ds(qid * NPAD + rbase, RC)])
            return 0
        lax.fori_loop(0, NRC, _g0, 0)

    plsc.subcore_barrier()

    def _pass(p, _):
        k = p // 2
        q = p % 2
        qid = 2 * c + q

        @pl.when(p >= 1)
        def _():
            delta = jnp.where(q == 1, NPAD, -NPAD).astype(jnp.int32)
            dvec = jnp.broadcast_to(delta, (LANES,))

            def _rb(j, _):
                for g in range(EC // LANES):
                    sl = pl.ds(g * LANES, LANES)
                    src_v[j, sl] = src_v[j, sl] + dvec
                return 0
            lax.fori_loop(0, NCHUNK, _rb, 0)

        for jj in range(RD):
            pltpu.async_copy(gflat.at[src_v.at[jj]],
                             gbufv.at[jj], sem.at[jj])

        def _edge(j, _):
            b = lax.rem(j, RD)
            bn = lax.rem(j + RD - 1, RD)

            @pl.when(j >= 1)
            def _():
                pltpu.make_async_copy(
                    gbufv.at[bn], acc.at[dst_v.at[j - 1]],
                    sem2.at[bn]).wait()

            @pl.when((j >= 1) & (j + RD - 1 < NCHUNK))
            def _():
                pltpu.async_copy(
                    gflat.at[src_v.at[j + RD - 1]],
                    gbufv.at[bn], sem.at[bn])

            pltpu.make_async_copy(
                gflat.at[src_v.at[j]], gbufv.at[b],
                sem.at[b]).wait()
            pltpu.async_copy(gbufv.at[b], acc.at[dst_v.at[j]], sem2.at[b],
                             add=True)
            return 0
        lax.fori_loop(0, NCHUNK, _edge, 0)
        pltpu.make_async_copy(
            gbufv.at[(NCHUNK - 1) % RD], acc.at[dst_v.at[NCHUNK - 1]],
            sem2.at[(NCHUNK - 1) % RD]).wait()

        plsc.subcore_barrier()

        def _scale(cc, _):
            rbase = nbase + cc * RC
            pltpu.sync_copy(acc.at[pl.ds(rbase, RC)], rowbuf)
            _scale_rows(cc * RC, True)
            pltpu.sync_copy(rowbuf, hops.at[k, qid, pl.ds(rbase, RC)])

            @pl.when(k < R_HOPS - 1)
            def _():
                pltpu.sync_copy(gout, gflat.at[pl.ds(qid * NPAD + rbase, RC)])
            _zero_acc(rbase)
            return 0
        lax.fori_loop(0, NRC, _scale, 0)

        plsc.subcore_barrier()
        return 0
    lax.fori_loop(0, 2 * R_HOPS, _pass, 0)


def _sc_propagate(x_qsplit, srcp, dstp):
    mesh = plsc.VectorSubcoreMesh(
        core_axis_name="c", subcore_axis_name="s",
        num_cores=2, num_subcores=NSUB)
    return pl.kernel(
        _sc_body,
        out_type=[
            jax.ShapeDtypeStruct((R_HOPS, 4, NPAD, DQ), jnp.float32),
            jax.ShapeDtypeStruct((4 * NPAD, DQ), jnp.float32),
            jax.ShapeDtypeStruct((2, NSUB, NPAD), jnp.float32),
        ],
        mesh=mesh,
        scratch_types=[
            pltpu.VMEM((NCHUNK, EC), jnp.int32),
            pltpu.VMEM((NCHUNK, EC), jnp.int32),
            pltpu.VMEM((RD, EC, DQ), jnp.float32),
            pltpu.VMEM((RC, DQ), jnp.float32),
            pltpu.VMEM((RC, DQ), jnp.float32),
            pltpu.VMEM((ZR, DQ), jnp.float32),
            pltpu.VMEM((NPAD,), jnp.float32),
            pltpu.VMEM((NSUB, RC), jnp.float32),
            pltpu.VMEM((RPT,), jnp.float32),
            pltpu.VMEM_SHARED((NPAD, DQ), jnp.float32),
            pltpu.SemaphoreType.DMA((RD,)),
            pltpu.SemaphoreType.DMA((RD,)),
        ],
        compiler_params=pltpu.CompilerParams(
            needs_layout_passes=False, use_tc_tiling_on_sc=False),
    )(x_qsplit, srcp, dstp)


def _mlp_body(x_ref, hops_ref, w1_ref, b1_ref, w2_ref, b2_ref, o_ref):
    acc = jnp.dot(x_ref[...], w1_ref[0:D, :],
                  preferred_element_type=jnp.float32)
    for r in range(R_HOPS):
        for qd in range(4):
            base = D + r * D + qd * DQ
            acc += jnp.dot(hops_ref[r, qd], w1_ref[base:base + DQ, :],
                           preferred_element_type=jnp.float32)
    hid = jnp.maximum(acc + b1_ref[...], 0.0)
    o_ref[...] = jnp.dot(hid, w2_ref[...],
                         preferred_element_type=jnp.float32) + b2_ref[...]


def _mlp(x_pad, hops, W1, b1, W2, b2):
    BR = 1024
    grid = (NPAD // BR,)
    return pl.pallas_call(
        _mlp_body,
        grid=grid,
        in_specs=[
            pl.BlockSpec((BR, D), lambda i: (i, 0)),
            pl.BlockSpec((R_HOPS, 4, BR, DQ), lambda i: (0, 0, i, 0)),
            pl.BlockSpec((D * (R_HOPS + 1), D), lambda i: (0, 0)),
            pl.BlockSpec((1, D), lambda i: (0, 0)),
            pl.BlockSpec((D, D), lambda i: (0, 0)),
            pl.BlockSpec((1, D), lambda i: (0, 0)),
        ],
        out_specs=pl.BlockSpec((BR, D), lambda i: (i, 0)),
        out_shape=jax.ShapeDtypeStruct((NPAD, D), jnp.float32),
        compiler_params=pltpu.CompilerParams(
            dimension_semantics=("arbitrary",),
        ),
    )(x_pad, hops, W1, b1, W2, b2)


@jax.jit
def kernel(x, edge_index, W1, b1, W2, b2):
    x_pad = jnp.pad(x, ((0, NPAD - N), (0, 0)))
    x_qsplit = x_pad.reshape(NPAD, 4, DQ).transpose(1, 0, 2)

    ei = edge_index.astype(jnp.int32)
    pad_e = NSUB * EPT - E
    src = jnp.pad(ei[0], (0, pad_e), constant_values=N)
    dst = jnp.pad(ei[1], (0, pad_e), constant_values=N)
    srcp = src.reshape(NSUB * NCHUNK, EC)
    dstp = dst.reshape(NSUB * NCHUNK, EC)

    hops, _g, _dp = _sc_propagate(x_qsplit, srcp, dstp)

    out = _mlp(x_pad, hops, W1, b1.reshape(1, D), W2, b2.reshape(1, D))
    return out[:N]
